# Initial kernel scaffold; baseline (speedup 1.0000x reference)
import jax
import jax.numpy as jnp
from jax.experimental import pallas as pl


def kernel(x, w1, b1, ln1_g, ln1_b, w2, b2, ln2_g, ln2_b):
    raise NotImplementedError("write your pallas kernel here")


# speedup vs baseline: 5.2231x; 5.2231x over previous
import functools

import jax
import jax.numpy as jnp
from jax import lax
from jax.experimental import pallas as pl
from jax.experimental.pallas import tpu as pltpu

_LN_EPS = 1e-5


def _fused_kernel(xp_ref, w1_ref, b1_ref, g1_ref, be1_ref, sel_ref, selt_ref,
                  w2_ref, b2_ref, g2_ref, be2_ref, o_ref):
    bt = xp_ref.shape[0]
    inv_n = 1.0 / 1024.0
    hi = lax.Precision.HIGHEST

    xk = xp_ref[...].reshape(bt * 128, 144)
    y = jnp.dot(xk, w1_ref[...], preferred_element_type=jnp.float32)
    y = y.reshape(bt, 128, 128) + b1_ref[...]

    s1 = jnp.sum(y, axis=1)
    s2 = jnp.sum(y * y, axis=1)
    stats = jnp.concatenate([s1, s2], axis=0)
    statc = jnp.dot(stats, sel_ref[...],
                    preferred_element_type=jnp.float32, precision=hi) * inv_n
    statb = jnp.dot(statc, selt_ref[...],
                    preferred_element_type=jnp.float32, precision=hi)
    mu = statb[:bt][:, None, :]
    var = jnp.maximum(statb[bt:][:, None, :] - mu * mu, 0.0)
    ya = (y - mu) * lax.rsqrt(var + _LN_EPS) * g1_ref[...] + be1_ref[...]
    ya = jnp.maximum(ya, 0.0).astype(jnp.bfloat16)

    z = jnp.dot(ya[:, 0:16, :].reshape(bt * 16, 128), w2_ref[0],
                preferred_element_type=jnp.float32)
    for lh in range(1, 8):
        z = z + jnp.dot(ya[:, lh * 16:(lh + 1) * 16, :].reshape(bt * 16, 128),
                        w2_ref[lh], preferred_element_type=jnp.float32)
    z = z + b2_ref[...]

    mu2 = jnp.mean(z, axis=-1, keepdims=True)
    d2 = z - mu2
    var2 = jnp.mean(d2 * d2, axis=-1, keepdims=True)
    zo = d2 * lax.rsqrt(var2 + _LN_EPS) * g2_ref[...] + be2_ref[...]
    o_ref[...] = jnp.maximum(zo, 0.0).reshape(bt, 16, 32)


@functools.partial(jax.jit, static_argnums=())
def kernel(x, w1, b1, ln1_g, ln1_b, w2, b2, ln2_g, ln2_b):
    B = x.shape[0]
    C1, Cin, KH, KW = w1.shape
    K1 = Cin * KH * KW

    x2 = x[:, :Cin].astype(jnp.float32)
    xpad = jnp.pad(x2, ((0, 0), (0, 0), (1, 1), (1, 1)))
    taps = []
    for cin in range(Cin):
        for kh in range(KH):
            for kw in range(KW):
                v = xpad[:, cin, kh:kh + 64:2, kw:kw + 64:2]
                taps.append(v)
    t = jnp.stack(taps, axis=1).astype(jnp.bfloat16)
    t = t.reshape(B, K1, 4, 8, 4, 8).transpose(0, 3, 2, 4, 1, 5)
    xp = t.reshape(B, 128, K1 * 8)

    w1_mat = w1.reshape(C1, K1).astype(jnp.float32)
    eye8 = jnp.eye(8, dtype=jnp.float32)
    w1p = (w1_mat.T[:, None, :, None] * eye8[None, :, None, :]).reshape(K1 * 8, 128)
    w1p = w1p.astype(jnp.bfloat16)
    b1r = jnp.repeat(b1.astype(jnp.float32), 8)[None, :]

    g1r = ln1_g.astype(jnp.float32).reshape(4, 8, 4, 8).transpose(1, 0, 2, 3).reshape(128, 8)
    be1r = ln1_b.astype(jnp.float32).reshape(4, 8, 4, 8).transpose(1, 0, 2, 3).reshape(128, 8)
    g1big = jnp.tile(g1r, (1, C1))
    be1big = jnp.tile(be1r, (1, C1))

    sel = jnp.repeat(jnp.eye(C1, dtype=jnp.float32), 8, axis=0)
    selt = sel.T

    H = w2.shape[0]
    w2s = w2.astype(jnp.float32).transpose(2, 1, 3, 0).reshape(8, 128, H)
    w2s = w2s.astype(jnp.bfloat16)
    b2r = b2.reshape(1, H).astype(jnp.float32)
    g2r = ln2_g.reshape(1, H).astype(jnp.float32)
    be2r = ln2_b.reshape(1, H).astype(jnp.float32)

    bt = 8
    while B % bt or (B // bt) < 2:
        bt //= 2
        if bt == 1:
            break

    out = pl.pallas_call(
        _fused_kernel,
        out_shape=jax.ShapeDtypeStruct((B, 16, H), jnp.float32),
        grid=(B // bt,),
        in_specs=[
            pl.BlockSpec((bt, 128, K1 * 8), lambda i: (i, 0, 0)),
            pl.BlockSpec((K1 * 8, 128), lambda i: (0, 0)),
            pl.BlockSpec((1, 128), lambda i: (0, 0)),
            pl.BlockSpec((128, 128), lambda i: (0, 0)),
            pl.BlockSpec((128, 128), lambda i: (0, 0)),
            pl.BlockSpec((128, C1), lambda i: (0, 0)),
            pl.BlockSpec((C1, 128), lambda i: (0, 0)),
            pl.BlockSpec((8, 128, H), lambda i: (0, 0, 0)),
            pl.BlockSpec((1, H), lambda i: (0, 0)),
            pl.BlockSpec((1, H), lambda i: (0, 0)),
            pl.BlockSpec((1, H), lambda i: (0, 0)),
        ],
        out_specs=pl.BlockSpec((bt, 16, H), lambda i: (i, 0, 0)),
        compiler_params=pltpu.CompilerParams(
            dimension_semantics=("parallel",),
            vmem_limit_bytes=64 * 1024 * 1024),
    )(xp, w1p, b1r, g1big, be1big, sel, selt, w2s, b2r, g2r, be2r)
    return out


# speedup vs baseline: 11.6750x; 2.2353x over previous
import jax
import jax.numpy as jnp
from jax import lax
from jax.experimental import pallas as pl
from jax.experimental.pallas import tpu as pltpu

_LN_EPS = 1e-5


def _fused_kernel(v_ref, w1_ref, b1_ref, g1_ref, be1_ref, sa_ref, sat_ref,
                  w2_ref, b2_ref, g2_ref, be2_ref, sb_ref, sbt_ref, o_ref):
    bt = v_ref.shape[0]
    hi = lax.Precision.HIGHEST

    y = jnp.dot(v_ref[...].reshape(bt * 32, 396), w1_ref[...],
                preferred_element_type=jnp.float32)
    y = y.reshape(bt, 32, 512) + b1_ref[...]

    s1 = jnp.sum(y, axis=1)
    s2 = jnp.sum(y * y, axis=1)
    stats = jnp.concatenate([s1, s2], axis=0)
    statc = jnp.dot(stats, sa_ref[...],
                    preferred_element_type=jnp.float32, precision=hi) * (1.0 / 1024.0)
    statb = jnp.dot(statc, sat_ref[...],
                    preferred_element_type=jnp.float32, precision=hi)
    mu = statb[:bt][:, None, :]
    var = jnp.maximum(statb[bt:][:, None, :] - mu * mu, 0.0)
    ya = (y - mu) * lax.rsqrt(var + _LN_EPS) * g1_ref[...] + be1_ref[...]
    ya = jnp.maximum(ya, 0.0).astype(jnp.bfloat16)

    z = jnp.dot(ya[:, 0:4, :].reshape(bt * 4, 512), w2_ref[0],
                preferred_element_type=jnp.float32)
    for lh in range(1, 8):
        z = z + jnp.dot(ya[:, lh * 4:(lh + 1) * 4, :].reshape(bt * 4, 512),
                        w2_ref[lh], preferred_element_type=jnp.float32)
    z = z + b2_ref[...]

    zst = jnp.concatenate([z, z * z], axis=0)
    zc = jnp.dot(zst, sb_ref[...],
                 preferred_element_type=jnp.float32, precision=hi) * (1.0 / 32.0)
    zb = jnp.dot(zc, sbt_ref[...],
                 preferred_element_type=jnp.float32, precision=hi)
    n = bt * 4
    mu2 = zb[:n]
    var2 = jnp.maximum(zb[n:] - mu2 * mu2, 0.0)
    zo = (z - mu2) * lax.rsqrt(var2 + _LN_EPS) * g2_ref[...] + be2_ref[...]
    o_ref[...] = jnp.maximum(zo, 0.0).reshape(bt, 4, 128)


def kernel(x, w1, b1, ln1_g, ln1_b, w2, b2, ln2_g, ln2_b):
    B = x.shape[0]
    C1, Cin, KH, KW = w1.shape

    x2 = x[:, :Cin].astype(jnp.float32)
    xpad = jnp.pad(x2, ((0, 0), (0, 0), (1, 1), (1, 1)))
    rows = []
    for cin in range(Cin):
        for kh in range(KH):
            rows.append(xpad[:, cin, kh:kh + 64:2, :])
    v = jnp.stack(rows, axis=2)
    v = (v.reshape(B, 4, 8, 6, 66).transpose(0, 2, 1, 3, 4)
         .reshape(B, 32, 396).astype(jnp.bfloat16))

    wcol = jnp.arange(66)[None, :, None]
    wo = jnp.arange(32)[None, None, :]
    kwi = jnp.arange(KW)[:, None, None]
    ek = (wcol == 2 * wo + kwi).astype(jnp.float32)
    w1v = jnp.einsum('cikj,jwo->ikwco', w1.astype(jnp.float32), ek)
    w1v = w1v.reshape(Cin * KH * 66, C1 * 32).astype(jnp.bfloat16)
    b1r = jnp.repeat(b1.astype(jnp.float32), 32)[None, :]

    g1p = ln1_g.astype(jnp.float32).reshape(4, 8, 32).transpose(1, 0, 2).reshape(32, 32)
    be1p = ln1_b.astype(jnp.float32).reshape(4, 8, 32).transpose(1, 0, 2).reshape(32, 32)
    g1v = jnp.tile(g1p, (1, C1))
    be1v = jnp.tile(be1p, (1, C1))

    sa = jnp.repeat(jnp.eye(C1, dtype=jnp.float32), 32, axis=0)
    sat = sa.T

    H = w2.shape[0]
    t2 = w2.astype(jnp.float32).transpose(2, 1, 3, 0)
    eye4 = jnp.eye(4, dtype=jnp.float32)
    w2v = jnp.einsum('lcwh,pq->lcpwqh', t2, eye4).reshape(8, 512, 4 * H)
    w2v = w2v.astype(jnp.bfloat16)
    b2r = jnp.tile(b2.astype(jnp.float32), 4)[None, :]
    g2r = jnp.tile(ln2_g.astype(jnp.float32), 4)[None, :]
    be2r = jnp.tile(ln2_b.astype(jnp.float32), 4)[None, :]

    sb = jnp.repeat(jnp.eye(4, dtype=jnp.float32), H, axis=0)
    sbt = sb.T

    bt = 16
    while B % bt or (B // bt) < 2:
        bt //= 2
        if bt <= 1:
            bt = 1
            break

    out = pl.pallas_call(
        _fused_kernel,
        out_shape=jax.ShapeDtypeStruct((B, 4, 4 * H), jnp.float32),
        grid=(B // bt,),
        in_specs=[
            pl.BlockSpec((bt, 32, 396), lambda i: (i, 0, 0)),
            pl.BlockSpec((396, 512), lambda i: (0, 0)),
            pl.BlockSpec((1, 512), lambda i: (0, 0)),
            pl.BlockSpec((32, 512), lambda i: (0, 0)),
            pl.BlockSpec((32, 512), lambda i: (0, 0)),
            pl.BlockSpec((512, C1), lambda i: (0, 0)),
            pl.BlockSpec((C1, 512), lambda i: (0, 0)),
            pl.BlockSpec((8, 512, 4 * H), lambda i: (0, 0, 0)),
            pl.BlockSpec((1, 4 * H), lambda i: (0, 0)),
            pl.BlockSpec((1, 4 * H), lambda i: (0, 0)),
            pl.BlockSpec((1, 4 * H), lambda i: (0, 0)),
            pl.BlockSpec((4 * H, 4), lambda i: (0, 0)),
            pl.BlockSpec((4, 4 * H), lambda i: (0, 0)),
        ],
        out_specs=pl.BlockSpec((bt, 4, 4 * H), lambda i: (i, 0, 0)),
        compiler_params=pltpu.CompilerParams(
            dimension_semantics=("parallel",),
            vmem_limit_bytes=64 * 1024 * 1024),
    )(v, w1v, b1r, g1v, be1v, sa, sat, w2v, b2r, g2r, be2r, sb, sbt)
    return out.reshape(B, 16, H)


# speedup vs baseline: 15.5981x; 1.3360x over previous
import jax
import jax.numpy as jnp
from jax import lax
from jax.experimental import pallas as pl
from jax.experimental.pallas import tpu as pltpu

_LN_EPS = 1e-5


def _fused_kernel(x_ref, w1_ref, b1_ref, g1_ref, be1_ref, sa_ref, sat_ref,
                  w2_ref, b2_ref, g2_ref, be2_ref, sb_ref, sbt_ref, o_ref):
    bt = x_ref.shape[0]
    hi = lax.Precision.HIGHEST

    xb = x_ref[...]
    zrow = jnp.zeros((bt, 1, 64), jnp.float32)
    groups = []
    for cin in range(2):
        ec = xb[:, cin, :, 0:64]
        oc = xb[:, cin, :, 64:128]
        g0 = jnp.concatenate([zrow, oc[:, :31, :]], axis=1)
        groups += [g0, ec, oc]
    v = jnp.concatenate(groups, axis=2).astype(jnp.bfloat16)

    y = jnp.dot(v.reshape(bt * 32, 384), w1_ref[...],
                preferred_element_type=jnp.float32)
    y = y.reshape(bt, 32, 512) + b1_ref[...]

    s1 = jnp.sum(y, axis=1)
    s2 = jnp.sum(y * y, axis=1)
    stats = jnp.concatenate([s1, s2], axis=0)
    statc = jnp.dot(stats, sa_ref[...],
                    preferred_element_type=jnp.float32, precision=hi) * (1.0 / 1024.0)
    statb = jnp.dot(statc, sat_ref[...],
                    preferred_element_type=jnp.float32, precision=hi)
    mu = statb[:bt][:, None, :]
    var = jnp.maximum(statb[bt:][:, None, :] - mu * mu, 0.0)
    ya = (y - mu) * lax.rsqrt(var + _LN_EPS) * g1_ref[...] + be1_ref[...]
    ya = jnp.maximum(ya, 0.0).astype(jnp.bfloat16)

    ya4 = ya.reshape(bt, 4, 8, 512)
    z = jnp.dot(ya4[:, :, 0, :].reshape(bt * 4, 512), w2_ref[0],
                preferred_element_type=jnp.float32)
    for lh in range(1, 8):
        z = z + jnp.dot(ya4[:, :, lh, :].reshape(bt * 4, 512), w2_ref[lh],
                        preferred_element_type=jnp.float32)
    z = z + b2_ref[...]

    zst = jnp.concatenate([z, z * z], axis=0)
    zc = jnp.dot(zst, sb_ref[...],
                 preferred_element_type=jnp.float32, precision=hi) * (1.0 / 32.0)
    zb = jnp.dot(zc, sbt_ref[...],
                 preferred_element_type=jnp.float32, precision=hi)
    n = bt * 4
    mu2 = zb[:n]
    var2 = jnp.maximum(zb[n:] - mu2 * mu2, 0.0)
    zo = (z - mu2) * lax.rsqrt(var2 + _LN_EPS) * g2_ref[...] + be2_ref[...]
    o_ref[...] = jnp.maximum(zo, 0.0).reshape(bt, 4, 128)


def kernel(x, w1, b1, ln1_g, ln1_b, w2, b2, ln2_g, ln2_b):
    B = x.shape[0]
    C1, Cin, KH, KW = w1.shape

    xr = x.reshape(B, x.shape[1], 32, 128)

    wcol = jnp.arange(64)[None, :, None]
    wo = jnp.arange(32)[None, None, :]
    kwi = jnp.arange(KW)[:, None, None]
    ek = (wcol == 2 * wo + kwi - 1).astype(jnp.float32)
    w1v = jnp.einsum('cikj,jwo->ikwco', w1.astype(jnp.float32), ek)
    w1v = w1v.reshape(Cin * KH * 64, C1 * 32).astype(jnp.bfloat16)
    b1r = jnp.repeat(b1.astype(jnp.float32), 32)[None, :]

    g1v = jnp.tile(ln1_g.astype(jnp.float32), (1, C1))
    be1v = jnp.tile(ln1_b.astype(jnp.float32), (1, C1))

    sa = jnp.repeat(jnp.eye(C1, dtype=jnp.float32), 32, axis=0)
    sat = sa.T

    H = w2.shape[0]
    t2 = w2.astype(jnp.float32).transpose(2, 1, 3, 0)
    eye4 = jnp.eye(4, dtype=jnp.float32)
    w2v = jnp.einsum('lcwh,pq->lcpwqh', t2, eye4).reshape(8, 512, 4 * H)
    w2v = w2v.astype(jnp.bfloat16)
    b2r = jnp.tile(b2.astype(jnp.float32), 4)[None, :]
    g2r = jnp.tile(ln2_g.astype(jnp.float32), 4)[None, :]
    be2r = jnp.tile(ln2_b.astype(jnp.float32), 4)[None, :]

    sb = jnp.repeat(jnp.eye(4, dtype=jnp.float32), H, axis=0)
    sbt = sb.T

    bt = 16
    while B % bt or (B // bt) < 2:
        bt //= 2
        if bt <= 1:
            bt = 1
            break

    out = pl.pallas_call(
        _fused_kernel,
        out_shape=jax.ShapeDtypeStruct((B, 4, 4 * H), jnp.float32),
        grid=(B // bt,),
        in_specs=[
            pl.BlockSpec((bt, Cin, 32, 128), lambda i: (i, 0, 0, 0)),
            pl.BlockSpec((384, 512), lambda i: (0, 0)),
            pl.BlockSpec((1, 512), lambda i: (0, 0)),
            pl.BlockSpec((32, 512), lambda i: (0, 0)),
            pl.BlockSpec((32, 512), lambda i: (0, 0)),
            pl.BlockSpec((512, C1), lambda i: (0, 0)),
            pl.BlockSpec((C1, 512), lambda i: (0, 0)),
            pl.BlockSpec((8, 512, 4 * H), lambda i: (0, 0, 0)),
            pl.BlockSpec((1, 4 * H), lambda i: (0, 0)),
            pl.BlockSpec((1, 4 * H), lambda i: (0, 0)),
            pl.BlockSpec((1, 4 * H), lambda i: (0, 0)),
            pl.BlockSpec((4 * H, 4), lambda i: (0, 0)),
            pl.BlockSpec((4, 4 * H), lambda i: (0, 0)),
        ],
        out_specs=pl.BlockSpec((bt, 4, 4 * H), lambda i: (i, 0, 0)),
        compiler_params=pltpu.CompilerParams(
            dimension_semantics=("parallel",),
            vmem_limit_bytes=64 * 1024 * 1024),
    )(xr, w1v, b1r, g1v, be1v, sa, sat, w2v, b2r, g2r, be2r, sb, sbt)
    return out.reshape(B, 16, H)


# speedup vs baseline: 21.0507x; 1.3496x over previous
import jax
import jax.numpy as jnp
from jax import lax
from jax.experimental import pallas as pl
from jax.experimental.pallas import tpu as pltpu

_LN_EPS = 1e-5


def _fused_kernel(x_ref, w1_ref, aff1_ref, pk_ref, sa_ref, sat_ref,
                  w2_ref, sb_ref, sbt_ref, o_ref, scr):
    bt = x_ref.shape[0]
    hi = lax.Precision.HIGHEST

    xb = x_ref[...]
    zrow = jnp.zeros((bt, 1, 64), jnp.float32)
    groups = []
    for cin in range(2):
        ec = xb[:, cin, :, 0:64]
        oc = xb[:, cin, :, 64:128]
        g0 = jnp.concatenate([zrow, oc[:, :31, :]], axis=1)
        groups += [g0, ec, oc]
    v = jnp.concatenate(groups, axis=2).astype(jnp.bfloat16)
    v2 = v.reshape(bt * 32, 384)

    b1r = pk_ref[0:1]
    ys = []
    s1 = jnp.zeros((bt, 128), jnp.float32)
    s2 = jnp.zeros((bt, 128), jnp.float32)
    for pw in range(4):
        ypw = jnp.dot(v2, w1_ref[pw], preferred_element_type=jnp.float32)
        ypw = ypw.reshape(bt, 32, 128) + b1r
        ys.append(ypw)
        s1 = s1 + jnp.sum(ypw, axis=1)
        s2 = s2 + jnp.sum(ypw * ypw, axis=1)

    stats = jnp.concatenate([s1, s2], axis=0)
    statc = jnp.dot(stats, sa_ref[...],
                    preferred_element_type=jnp.float32, precision=hi) * (1.0 / 1024.0)
    statb = jnp.dot(statc, sat_ref[...],
                    preferred_element_type=jnp.float32, precision=hi)
    mu = statb[:bt][:, None, :]
    var = jnp.maximum(statb[bt:][:, None, :] - mu * mu, 0.0)
    rs = lax.rsqrt(var + _LN_EPS)
    for pw in range(4):
        ya = (ys[pw] - mu) * rs * aff1_ref[0, pw] + aff1_ref[1, pw]
        scr[pw] = jnp.maximum(ya, 0.0).reshape(bt * 32, 128)

    zparts = []
    for pw in range(4):
        acc = jnp.dot(scr[pw, 0::8, :].astype(jnp.bfloat16), w2_ref[0],
                      preferred_element_type=jnp.float32)
        for lh in range(1, 8):
            acc = acc + jnp.dot(scr[pw, lh::8, :].astype(jnp.bfloat16),
                                w2_ref[lh], preferred_element_type=jnp.float32)
        zparts.append(acc)
    z = jnp.concatenate(zparts, axis=1) + pk_ref[1:2]

    zst = jnp.concatenate([z, z * z], axis=0)
    zc = jnp.dot(zst, sb_ref[...],
                 preferred_element_type=jnp.float32, precision=hi) * (1.0 / 32.0)
    zb = jnp.dot(zc, sbt_ref[...],
                 preferred_element_type=jnp.float32, precision=hi)
    n = bt * 4
    mu2 = zb[:n]
    var2 = jnp.maximum(zb[n:] - mu2 * mu2, 0.0)
    zo = (z - mu2) * lax.rsqrt(var2 + _LN_EPS) * pk_ref[2:3] + pk_ref[3:4]
    o_ref[...] = jnp.maximum(zo, 0.0).reshape(bt, 4, 128)


def kernel(x, w1, b1, ln1_g, ln1_b, w2, b2, ln2_g, ln2_b):
    B = x.shape[0]
    C1, Cin, KH, KW = w1.shape

    xr = x.reshape(B, x.shape[1], 32, 128)

    wcol = jnp.arange(64)[None, :, None]
    wo = jnp.arange(32)[None, None, :]
    kwi = jnp.arange(KW)[:, None, None]
    ek = (wcol == 2 * wo + kwi - 1).astype(jnp.float32)
    w1v = jnp.einsum('cikj,jwo->ikwco', w1.astype(jnp.float32), ek)
    w1v4 = (w1v.reshape(Cin, KH, 64, C1, 4, 8).transpose(4, 0, 1, 2, 3, 5)
            .reshape(4, Cin * KH * 64, C1 * 8).astype(jnp.bfloat16))

    def _aff(a):
        t = a.astype(jnp.float32).reshape(32, 4, 8).transpose(1, 0, 2)
        return jnp.tile(t[:, :, None, :], (1, 1, C1, 1)).reshape(4, 32, 128)
    aff1 = jnp.stack([_aff(ln1_g), _aff(ln1_b)])

    H = w2.shape[0]
    pk = jnp.stack([
        jnp.repeat(b1.astype(jnp.float32), 8),
        jnp.tile(b2.astype(jnp.float32), 4),
        jnp.tile(ln2_g.astype(jnp.float32), 4),
        jnp.tile(ln2_b.astype(jnp.float32), 4),
    ])

    sa = jnp.repeat(jnp.eye(C1, dtype=jnp.float32), 8, axis=0)
    sat = sa.T
    sb = jnp.repeat(jnp.eye(4, dtype=jnp.float32), H, axis=0)
    sbt = sb.T

    w2c = (w2.astype(jnp.float32).transpose(2, 1, 3, 0)
           .reshape(8, 128, H).astype(jnp.bfloat16))

    bt = 32
    while B % bt or (B // bt) < 2:
        bt //= 2
        if bt <= 1:
            bt = 1
            break

    out = pl.pallas_call(
        _fused_kernel,
        out_shape=jax.ShapeDtypeStruct((B, 4, 4 * H), jnp.float32),
        grid=(B // bt,),
        in_specs=[
            pl.BlockSpec((bt, Cin, 32, 128), lambda i: (i, 0, 0, 0)),
            pl.BlockSpec((4, 384, 128), lambda i: (0, 0, 0)),
            pl.BlockSpec((2, 4, 32, 128), lambda i: (0, 0, 0, 0)),
            pl.BlockSpec((4, 128), lambda i: (0, 0)),
            pl.BlockSpec((128, C1), lambda i: (0, 0)),
            pl.BlockSpec((C1, 128), lambda i: (0, 0)),
            pl.BlockSpec((8, 128, H), lambda i: (0, 0, 0)),
            pl.BlockSpec((128, 4), lambda i: (0, 0)),
            pl.BlockSpec((4, 128), lambda i: (0, 0)),
        ],
        out_specs=pl.BlockSpec((bt, 4, 4 * H), lambda i: (i, 0, 0)),
        scratch_shapes=[pltpu.VMEM((4, bt * 32, 128), jnp.float32)],
        compiler_params=pltpu.CompilerParams(
            dimension_semantics=("parallel",),
            vmem_limit_bytes=64 * 1024 * 1024),
    )(xr, w1v4, aff1, pk, sa, sat, w2c, sb, sbt)
    return out.reshape(B, 16, H)
